# baseline (speedup 1.0000x reference)
import jax
import jax.numpy as jnp
from jax.experimental import pallas as pl

_BLOCK = 5000


def _fused_mlp_kernel(x_ref, w0_ref, b0_ref, w1_ref, b1_ref, w2_ref, b2_ref,
                      w3_ref, b3_ref, out_ref):
    h = x_ref[...]
    for w_ref, b_ref in ((w0_ref, b0_ref), (w1_ref, b1_ref), (w2_ref, b2_ref)):
        h = jnp.dot(h, w_ref[...], preferred_element_type=jnp.float32) + b_ref[...]
        h = h * jax.nn.sigmoid(h)
    o = jnp.dot(h, w3_ref[...], preferred_element_type=jnp.float32) + b3_ref[...]
    m = jnp.max(o, axis=-1, keepdims=True)
    s = o - m
    lse = jnp.log(jnp.sum(jnp.exp(s), axis=-1, keepdims=True))
    out_ref[...] = s - lse


def kernel(x, edge_index, W0, b0, W1, b1, W2, b2, W3, b3):
    del edge_index
    n, d = x.shape
    n_out = W3.shape[1]
    grid = (n // _BLOCK,)

    def full(arr):
        return pl.BlockSpec(arr.shape, lambda i: (0,) * arr.ndim)

    b0r, b1r, b2r, b3r = (b.reshape(1, -1) for b in (b0, b1, b2, b3))
    return pl.pallas_call(
        _fused_mlp_kernel,
        grid=grid,
        in_specs=[
            pl.BlockSpec((_BLOCK, d), lambda i: (i, 0)),
            full(W0), full(b0r), full(W1), full(b1r),
            full(W2), full(b2r), full(W3), full(b3r),
        ],
        out_specs=pl.BlockSpec((_BLOCK, n_out), lambda i: (i, 0)),
        out_shape=jax.ShapeDtypeStruct((n, n_out), x.dtype),
    )(x, W0, b0r, W1, b1r, W2, b2r, W3, b3r)


# speedup vs baseline: 1.0444x; 1.0444x over previous
import jax
import jax.numpy as jnp
from jax.experimental import pallas as pl

_BLOCK = 2000


def _fused_mlp_kernel(x_ref, w0_ref, b0_ref, w1_ref, b1_ref, w2_ref, b2_ref,
                      w3_ref, b3_ref, out_ref):
    h = x_ref[...]
    for w_ref, b_ref in ((w0_ref, b0_ref), (w1_ref, b1_ref), (w2_ref, b2_ref)):
        h = jnp.dot(h, w_ref[...], preferred_element_type=jnp.float32) + b_ref[...]
        h = 0.5 * h * (1.0 + jnp.tanh(0.5 * h))
    o = jnp.dot(h, w3_ref[...], preferred_element_type=jnp.float32) + b3_ref[...]
    m = jnp.max(o, axis=-1, keepdims=True)
    s = o - m
    lse = jnp.log(jnp.sum(jnp.exp(s), axis=-1, keepdims=True))
    out_ref[...] = s - lse


def kernel(x, edge_index, W0, b0, W1, b1, W2, b2, W3, b3):
    del edge_index
    n, d = x.shape
    n_out = W3.shape[1]
    grid = (n // _BLOCK,)

    def full(arr):
        return pl.BlockSpec(arr.shape, lambda i: (0,) * arr.ndim)

    b0r, b1r, b2r, b3r = (b.reshape(1, -1) for b in (b0, b1, b2, b3))
    return pl.pallas_call(
        _fused_mlp_kernel,
        grid=grid,
        in_specs=[
            pl.BlockSpec((_BLOCK, d), lambda i: (i, 0)),
            full(W0), full(b0r), full(W1), full(b1r),
            full(W2), full(b2r), full(W3), full(b3r),
        ],
        out_specs=pl.BlockSpec((_BLOCK, n_out), lambda i: (i, 0)),
        out_shape=jax.ShapeDtypeStruct((n, n_out), x.dtype),
    )(x, W0, b0r, W1, b1r, W2, b2r, W3, b3r)
